# baseline (speedup 1.0000x reference)
import functools

import jax
import jax.numpy as jnp
from jax import lax
from jax.experimental import pallas as pl
from jax.experimental.pallas import tpu as pltpu
from jax.experimental.pallas import tpu_sc as plsc

_BATCH = 16384
_D = 64
_NC = 2
_NS = 16
_NW = _NC * _NS
_BPW = _BATCH // _NW
_CHUNK = 128
_NCHUNK = _BPW // _CHUNK
_LANES = 16
_PAIR_W = 2 * _D

_mesh = plsc.VectorSubcoreMesh(core_axis_name="c", subcore_axis_name="s")


@functools.partial(
    pl.kernel,
    out_type=jax.ShapeDtypeStruct((_NW, _LANES), jnp.float32),
    mesh=_mesh,
    scratch_types=[
        pltpu.VMEM((_BPW + _LANES,), jnp.int32),
        pltpu.VMEM((_NCHUNK, _CHUNK), jnp.int32),
        pltpu.VMEM((_CHUNK, _PAIR_W), jnp.float32),
        pltpu.VMEM((_CHUNK, _PAIR_W), jnp.float32),
        pltpu.VMEM((_BPW, _D), jnp.float32),
        pltpu.VMEM((_LANES,), jnp.float32),
        pltpu.SemaphoreType.DMA,
        pltpu.SemaphoreType.DMA,
    ],
)
def _center_loss_partials(feat_hbm, lab_hbm, cent_hbm, out_hbm,
                          lab_v, idx_v, cent0_v, cent1_v, feat_v,
                          acc_v, sem0, sem1):
    wid = lax.axis_index("s") * _NC + lax.axis_index("c")
    base = wid * _BPW

    pltpu.sync_copy(lab_hbm.at[wid], lab_v.at[pl.ds(0, _BPW)])
    for k in range(_NCHUNK):
        for t in range(_CHUNK // _LANES):
            lv = lab_v[pl.ds(k * _CHUNK + t * _LANES, _LANES)]
            idx_v[k, pl.ds(t * _LANES, _LANES)] = lv >> 1

    bufs = (cent0_v, cent1_v)
    sems = (sem0, sem1)
    gathers = [None] * _NCHUNK
    for k in range(2):
        gathers[k] = pltpu.async_copy(cent_hbm.at[idx_v.at[k]],
                                      bufs[k], sems[k])
    pltpu.sync_copy(feat_hbm.at[pl.ds(base, _BPW)], feat_v)

    ngrp = _D // _LANES
    accs = tuple(jnp.zeros((_LANES,), jnp.float32) for _ in range(ngrp))
    for k in range(_NCHUNK):
        gathers[k].wait()
        cb = bufs[k % 2]

        def row(i, accs, k=k, cb=cb):
            r = k * _CHUNK + i
            off = (lab_v[pl.ds(r, _LANES)][0] & 1) * _D
            out = []
            for j in range(ngrp):
                f = feat_v[r, pl.ds(j * _LANES, _LANES)]
                c = cb[i, pl.ds(off + j * _LANES, _LANES)]
                d = f - c
                out.append(accs[j] + d * d)
            return tuple(out)

        accs = lax.fori_loop(0, _CHUNK, row, accs)
        if k + 2 < _NCHUNK:
            gathers[k + 2] = pltpu.async_copy(cent_hbm.at[idx_v.at[k + 2]],
                                              bufs[k % 2], sems[k % 2])

    acc_v[...] = (accs[0] + accs[1]) + (accs[2] + accs[3])
    pltpu.sync_copy(acc_v, out_hbm.at[wid])


def kernel(features, labels, centers):
    labels = labels.astype(jnp.int32).reshape(_NW, _BPW)
    centers_pairs = centers.reshape(centers.shape[0] // 2, _PAIR_W)
    partials = _center_loss_partials(features, labels, centers_pairs)
    return jnp.sum(partials) / jnp.float32(_BATCH)


# speedup vs baseline: 1.2289x; 1.2289x over previous
import functools

import jax
import jax.numpy as jnp
from jax import lax
from jax.experimental import pallas as pl
from jax.experimental.pallas import tpu as pltpu
from jax.experimental.pallas import tpu_sc as plsc

_BATCH = 16384
_D = 64
_NC = 2
_NS = 16
_NW = _NC * _NS
_BPW = _BATCH // _NW
_CHUNK = 128
_NCHUNK = _BPW // _CHUNK
_LANES = 16

_mesh = plsc.VectorSubcoreMesh(core_axis_name="c", subcore_axis_name="s")


@functools.partial(
    pl.kernel,
    out_type=jax.ShapeDtypeStruct((_NW, _LANES), jnp.float32),
    mesh=_mesh,
    scratch_types=[
        pltpu.VMEM((_BPW + _LANES,), jnp.int32),
        pltpu.VMEM((_CHUNK, _D), jnp.float32),
        pltpu.VMEM((_CHUNK, _D), jnp.float32),
        pltpu.VMEM((_BPW, _D), jnp.float32),
        pltpu.VMEM((_LANES,), jnp.float32),
        pltpu.SemaphoreType.DMA,
        pltpu.SemaphoreType.DMA,
    ],
)
def _center_loss_partials(feat_hbm, lab_hbm, cent_hbm, out_hbm,
                          lab_v, cent0_v, cent1_v, feat_v,
                          acc_v, sem0, sem1):
    wid = lax.axis_index("s") * _NC + lax.axis_index("c")
    base = wid * _BPW

    pltpu.sync_copy(lab_hbm.at[wid], lab_v.at[pl.ds(0, _BPW)])

    bufs = (cent0_v, cent1_v)
    sems = (sem0, sem1)

    def fire_chunk(k):
        cb = bufs[k % 2]
        sem = sems[k % 2]

        def enqueue(i, _):
            lab = lab_v[pl.ds(k * _CHUNK + i, _LANES)][0]
            pltpu.async_copy(cent_hbm.at[lab], cb.at[i], sem)
            return 0

        lax.fori_loop(0, _CHUNK, enqueue, 0)

    def drain_chunk(k):
        pltpu.make_async_copy(cent_hbm.at[pl.ds(0, _CHUNK)],
                              bufs[k % 2], sems[k % 2]).wait()

    fire_chunk(0)
    fire_chunk(1)
    pltpu.sync_copy(feat_hbm.at[pl.ds(base, _BPW)], feat_v)

    ngrp = _D // _LANES
    accs = tuple(jnp.zeros((_LANES,), jnp.float32) for _ in range(ngrp))
    for k in range(_NCHUNK):
        drain_chunk(k)
        cb = bufs[k % 2]

        def row(i, accs, k=k, cb=cb):
            r = k * _CHUNK + i
            out = []
            for j in range(ngrp):
                f = feat_v[r, pl.ds(j * _LANES, _LANES)]
                c = cb[i, pl.ds(j * _LANES, _LANES)]
                d = f - c
                out.append(accs[j] + d * d)
            return tuple(out)

        accs = lax.fori_loop(0, _CHUNK, row, accs)
        if k + 2 < _NCHUNK:
            fire_chunk(k + 2)

    acc_v[...] = (accs[0] + accs[1]) + (accs[2] + accs[3])
    pltpu.sync_copy(acc_v, out_hbm.at[wid])


def kernel(features, labels, centers):
    labels = labels.astype(jnp.int32).reshape(_NW, _BPW)
    partials = _center_loss_partials(features, labels, centers)
    return jnp.sum(partials) / jnp.float32(_BATCH)


# speedup vs baseline: 2.1744x; 1.7695x over previous
import functools

import jax
import jax.numpy as jnp
from jax import lax
from jax.experimental import pallas as pl
from jax.experimental.pallas import tpu as pltpu
from jax.experimental.pallas import tpu_sc as plsc

_BATCH = 16384
_D = 64
_CLS = 100000
_NC = 2
_NS = 16
_NW = _NC * _NS
_WAVES = _D // _NW
_LANES = 16
_FCHUNK = 8192
_NFC = _BATCH // _FCHUNK
_UNROLL = 4

_mesh = plsc.VectorSubcoreMesh(core_axis_name="c", subcore_axis_name="s")


@functools.partial(
    pl.kernel,
    out_type=jax.ShapeDtypeStruct((_NW, _LANES), jnp.float32),
    mesh=_mesh,
    scratch_types=[
        pltpu.VMEM((_CLS,), jnp.float32),
        pltpu.VMEM((_BATCH,), jnp.int32),
        pltpu.VMEM((_FCHUNK,), jnp.float32),
        pltpu.VMEM((_LANES,), jnp.float32),
        pltpu.SemaphoreType.DMA,
    ],
    compiler_params=pltpu.CompilerParams(needs_layout_passes=False),
)
def _center_loss_partials(feat_hbm, lab_hbm, cent_hbm, out_hbm,
                          crow_v, lab_v, fbuf_v, acc_v, sem):
    wid = lax.axis_index("s") * _NC + lax.axis_index("c")

    pltpu.sync_copy(lab_hbm, lab_v)

    acc = jnp.zeros((_LANES,), jnp.float32)
    for w in range(_WAVES):
        d = w * _NW + wid
        cp = pltpu.async_copy(cent_hbm.at[d], crow_v, sem)
        pltpu.sync_copy(feat_hbm.at[d, pl.ds(0, _FCHUNK)], fbuf_v)
        cp.wait()
        for h in range(_NFC):
            if h > 0:
                pltpu.sync_copy(
                    feat_hbm.at[d, pl.ds(h * _FCHUNK, _FCHUNK)], fbuf_v)
            hbase = h * _FCHUNK

            def blk(i, acc, hbase=hbase):
                for u in range(_UNROLL):
                    o = (i * _UNROLL + u) * _LANES
                    idx = lab_v[pl.ds(hbase + o, _LANES)]
                    c = plsc.load_gather(crow_v, [idx])
                    f = fbuf_v[pl.ds(o, _LANES)]
                    df = f - c
                    acc = acc + df * df
                return acc

            acc = lax.fori_loop(0, _FCHUNK // (_LANES * _UNROLL), blk, acc)

    acc_v[...] = acc
    pltpu.sync_copy(acc_v, out_hbm.at[wid])


def kernel(features, labels, centers):
    labels = labels.astype(jnp.int32)
    partials = _center_loss_partials(features.T, labels, centers.T)
    return jnp.sum(partials) / jnp.float32(_BATCH)
